# baseline (speedup 1.0000x reference)
import functools

import jax
import jax.numpy as jnp
from jax.experimental import pallas as pl
from jax.experimental.pallas import tpu as pltpu


def _encoder_kernel(x_ref, w1_ref, b1_ref, w2_ref, b2_ref, o_ref, *, H, W, C):
    bf16 = jnp.bfloat16
    x2d = x_ref[...].reshape(H * W, C)
    xs1 = jnp.concatenate([x2d[1:], x2d[:1]], axis=0)
    xs2 = jnp.concatenate([x2d[2:], x2d[:2]], axis=0)
    xp = jnp.concatenate(
        [x2d.astype(bf16), xs1.astype(bf16), xs2.astype(bf16)], axis=1)

    z1 = jnp.dot(xp, w1_ref[...], preferred_element_type=jnp.float32)

    M1 = (H - 2) * W
    h = (z1[0:M1, 0:C] + z1[W:M1 + W, C:2 * C]
         + z1[2 * W:M1 + 2 * W, 2 * C:3 * C] + b1_ref[...])
    h = jnp.maximum(h, 0.0)

    hs1 = jnp.concatenate([h[1:], h[:1]], axis=0)
    hs2 = jnp.concatenate([h[2:], h[:2]], axis=0)
    hp = jnp.concatenate(
        [h.astype(bf16), hs1.astype(bf16), hs2.astype(bf16)], axis=1)

    z2 = jnp.dot(hp, w2_ref[...], preferred_element_type=jnp.float32)

    M2 = (H - 4) * W
    y = (z2[0:M2, 0:C] + z2[W:M2 + W, C:2 * C]
         + z2[2 * W:M2 + 2 * W, 2 * C:3 * C] + b2_ref[...])
    y = jnp.maximum(y, 0.0)
    y = y + xs2[2 * W:2 * W + M2, :]
    o_ref[...] = y.reshape(H - 4, W, C)[:, 0:W - 4, :]


def kernel(x, w1, b1, w2, b2):
    N, C, H, W = x.shape
    bf16 = jnp.bfloat16
    w1c = jnp.transpose(w1, (3, 1, 2, 0)).reshape(3 * C, 3 * C).astype(bf16)
    w2c = jnp.transpose(w2, (3, 1, 2, 0)).reshape(3 * C, 3 * C).astype(bf16)
    b1k = b1.reshape(1, C).astype(jnp.float32)
    b2k = b2.reshape(1, C).astype(jnp.float32)

    body = functools.partial(_encoder_kernel, H=H, W=W, C=C)

    def half(xs):
        nb = xs.shape[0]
        xh = jnp.transpose(xs, (0, 2, 3, 1)).astype(jnp.float32)
        out = pl.pallas_call(
            body,
            out_shape=jax.ShapeDtypeStruct((nb, H - 4, W - 4, C), jnp.float32),
            grid=(nb,),
            in_specs=[
                pl.BlockSpec((None, H, W, C), lambda b: (b, 0, 0, 0)),
                pl.BlockSpec((3 * C, 3 * C), lambda b: (0, 0)),
                pl.BlockSpec((1, C), lambda b: (0, 0)),
                pl.BlockSpec((3 * C, 3 * C), lambda b: (0, 0)),
                pl.BlockSpec((1, C), lambda b: (0, 0)),
            ],
            out_specs=pl.BlockSpec((None, H - 4, W - 4, C),
                                   lambda b: (b, 0, 0, 0)),
            compiler_params=pltpu.CompilerParams(
                dimension_semantics=("parallel",),
                vmem_limit_bytes=64 * 1024 * 1024),
        )(xh, w1c, b1k, w2c, b2k)
        return jnp.transpose(out, (0, 3, 1, 2))

    nh = N // 2
    return jnp.concatenate([half(x[:nh]), half(x[nh:])], axis=0)


# speedup vs baseline: 1.4400x; 1.4400x over previous
import functools

import jax
import jax.numpy as jnp
from jax.experimental import pallas as pl
from jax.experimental.pallas import tpu as pltpu


def _encoder_kernel(x_ref, w1_ref, b1_ref, w2_ref, b2_ref, o_ref, *, H, W, C):
    bf16 = jnp.bfloat16
    x2d = x_ref[...].reshape(H * W, C)
    xs1 = jnp.concatenate([x2d[1:], x2d[:1]], axis=0)
    xs2 = jnp.concatenate([x2d[2:], x2d[:2]], axis=0)
    xp = jnp.concatenate(
        [x2d.astype(bf16), xs1.astype(bf16), xs2.astype(bf16)], axis=1)

    z1 = jnp.dot(xp, w1_ref[...], preferred_element_type=jnp.float32)

    M1 = (H - 2) * W
    h = (z1[0:M1, 0:C] + z1[W:M1 + W, C:2 * C]
         + z1[2 * W:M1 + 2 * W, 2 * C:3 * C] + b1_ref[...])
    h = jnp.maximum(h, 0.0)

    hs1 = jnp.concatenate([h[1:], h[:1]], axis=0)
    hs2 = jnp.concatenate([h[2:], h[:2]], axis=0)
    hp = jnp.concatenate(
        [h.astype(bf16), hs1.astype(bf16), hs2.astype(bf16)], axis=1)

    z2 = jnp.dot(hp, w2_ref[...], preferred_element_type=jnp.float32)

    M2 = (H - 4) * W
    y = (z2[0:M2, 0:C] + z2[W:M2 + W, C:2 * C]
         + z2[2 * W:M2 + 2 * W, 2 * C:3 * C] + b2_ref[...])
    y = jnp.maximum(y, 0.0)
    y = y + xs2[2 * W:2 * W + M2, :]
    o_ref[...] = y.reshape(H - 4, W, C)[:, 0:W - 4, :]


def kernel(x, w1, b1, w2, b2):
    N, C, H, W = x.shape
    bf16 = jnp.bfloat16
    xh = jnp.transpose(x, (0, 2, 3, 1)).astype(jnp.float32)
    w1c = jnp.transpose(w1, (3, 1, 2, 0)).reshape(3 * C, 3 * C).astype(bf16)
    w2c = jnp.transpose(w2, (3, 1, 2, 0)).reshape(3 * C, 3 * C).astype(bf16)
    b1k = b1.reshape(1, C).astype(jnp.float32)
    b2k = b2.reshape(1, C).astype(jnp.float32)

    body = functools.partial(_encoder_kernel, H=H, W=W, C=C)
    out = pl.pallas_call(
        body,
        out_shape=jax.ShapeDtypeStruct((N, H - 4, W - 4, C), jnp.float32),
        grid=(N,),
        in_specs=[
            pl.BlockSpec((None, H, W, C), lambda b: (b, 0, 0, 0)),
            pl.BlockSpec((3 * C, 3 * C), lambda b: (0, 0)),
            pl.BlockSpec((1, C), lambda b: (0, 0)),
            pl.BlockSpec((3 * C, 3 * C), lambda b: (0, 0)),
            pl.BlockSpec((1, C), lambda b: (0, 0)),
        ],
        out_specs=pl.BlockSpec((None, H - 4, W - 4, C), lambda b: (b, 0, 0, 0)),
        compiler_params=pltpu.CompilerParams(
            dimension_semantics=("parallel",),
            vmem_limit_bytes=64 * 1024 * 1024),
    )(xh, w1c, b1k, w2c, b2k)
    return jnp.transpose(out, (0, 3, 1, 2))
